# baseline (speedup 1.0000x reference)
import jax
import jax.numpy as jnp
from jax import lax
from jax.experimental import pallas as pl
from jax.experimental.pallas import tpu as pltpu
from jax.experimental.pallas import tpu_sc as plsc

V = 1000000
D = 64
B = 16384
NEG = 20
NP = NEG + 1

NC = 2
NS = 16
NW = NC * NS
B_PER_W = B // NW
BLK = 32
NBLK = B_PER_W // BLK
NG = BLK // 16


def _sc_body(c2_hbm, ch_hbm, x2_hbm, xh_hbm, n2_hbm, nh_hbm, in_hbm, out_hbm,
             pos_hbm, negT_hbm,
             c2v, chv, x2v, xhv, n2v, nhv, crows, prows, scores, sem):
    wid = lax.axis_index("s") * NC + lax.axis_index("c")
    wbase = wid * B_PER_W
    lanes = lax.iota(jnp.int32, 16)

    pltpu.sync_copy(c2_hbm.at[pl.ds(wbase, B_PER_W)], c2v)
    pltpu.sync_copy(ch_hbm.at[pl.ds(wbase, B_PER_W)], chv)
    pltpu.sync_copy(x2_hbm.at[pl.ds(wbase, B_PER_W)], x2v)
    pltpu.sync_copy(xh_hbm.at[pl.ds(wbase, B_PER_W)], xhv)
    for j in range(NEG):
        pltpu.sync_copy(n2_hbm.at[j, pl.ds(wbase, B_PER_W)], n2v.at[j])
        pltpu.sync_copy(nh_hbm.at[j, pl.ds(wbase, B_PER_W)], nhv.at[j])

    def blk_body(sb, _):
        off = sb * BLK
        descs = [
            pltpu.async_copy(in_hbm.at[c2v.at[pl.ds(off, BLK)]], crows, sem),
            pltpu.async_copy(out_hbm.at[x2v.at[pl.ds(off, BLK)]], prows.at[0], sem),
        ]
        for j in range(NEG):
            descs.append(
                pltpu.async_copy(
                    out_hbm.at[n2v.at[j, pl.ds(off, BLK)]], prows.at[1 + j], sem
                )
            )
        for dsc in descs:
            dsc.wait()

        def group_body(g, _):
            rid = g * 16 + lanes
            chg = chv[pl.ds(off + g * 16, 16)]
            phg = [xhv[pl.ds(off + g * 16, 16)]] + [
                nhv[j, pl.ds(off + g * 16, 16)] for j in range(NEG)
            ]

            def d_body(d, accs):
                dv = jnp.broadcast_to(d, (16,))
                cv = plsc.load_gather(crows, [rid, chg + dv])
                return tuple(
                    acc
                    + cv
                    * plsc.load_gather(
                        prows, [jnp.full((16,), t, jnp.int32), rid, phg[t] + dv]
                    )
                    for t, acc in enumerate(accs)
                )

            accs = lax.fori_loop(
                0, D, d_body, tuple(jnp.zeros((16,), jnp.float32) for _ in range(NP))
            )
            for t in range(NP):
                scores[t, pl.ds(g * 16, 16)] = accs[t]
            return _

        lax.fori_loop(0, NG, group_body, None)

        pltpu.sync_copy(scores.at[0], pos_hbm.at[pl.ds(wbase + off, BLK)])
        for j in range(NEG):
            pltpu.sync_copy(scores.at[1 + j], negT_hbm.at[j, pl.ds(wbase + off, BLK)])
        return _

    lax.fori_loop(0, NBLK, blk_body, None)


@jax.jit
def _sc_scores(c2, ch, x2, xh, n2T, nhT, in_r, out_r):
    mesh = plsc.VectorSubcoreMesh(
        core_axis_name="c", subcore_axis_name="s", num_cores=NC, num_subcores=NS
    )
    f = pl.kernel(
        _sc_body,
        out_type=(
            jax.ShapeDtypeStruct((B,), jnp.float32),
            jax.ShapeDtypeStruct((NEG, B), jnp.float32),
        ),
        mesh=mesh,
        compiler_params=pltpu.CompilerParams(needs_layout_passes=False),
        scratch_types=[
            pltpu.VMEM((B_PER_W,), jnp.int32),
            pltpu.VMEM((B_PER_W,), jnp.int32),
            pltpu.VMEM((B_PER_W,), jnp.int32),
            pltpu.VMEM((B_PER_W,), jnp.int32),
            pltpu.VMEM((NEG, B_PER_W), jnp.int32),
            pltpu.VMEM((NEG, B_PER_W), jnp.int32),
            pltpu.VMEM((BLK, 2 * D), jnp.float32),
            pltpu.VMEM((NP, BLK, 2 * D), jnp.float32),
            pltpu.VMEM((NP, BLK), jnp.float32),
            pltpu.SemaphoreType.DMA,
        ],
    )
    return f(c2, ch, x2, xh, n2T, nhT, in_r, out_r)


def _loss_body(pos_ref, neg_ref, out_ref):
    p = pos_ref[...]
    n = neg_ref[...]
    total = jnp.sum(jax.nn.log_sigmoid(p)) + jnp.sum(jax.nn.log_sigmoid(-n))
    out_ref[...] = jnp.reshape(-total / B, (1, 1))


@jax.jit
def _tc_loss(pos, neg):
    out = pl.pallas_call(
        _loss_body,
        out_shape=jax.ShapeDtypeStruct((1, 1), jnp.float32),
    )(pos.reshape(128, 128), neg.reshape(NEG * B // 128, 128))
    return out[0, 0]


def kernel(center_words, context_words, negative_words, in_emb, out_emb):
    in_r = in_emb.reshape(V // 2, 2 * D)
    out_r = out_emb.reshape(V // 2, 2 * D)
    c2 = center_words >> 1
    ch = (center_words & 1) << 6
    x2 = context_words >> 1
    xh = (context_words & 1) << 6
    negT = negative_words.T
    n2T = negT >> 1
    nhT = (negT & 1) << 6
    pos, negs = _sc_scores(c2, ch, x2, xh, n2T, nhT, in_r, out_r)
    return _tc_loss(pos, negs)
